# baseline (speedup 1.0000x reference)
import math

import jax
import jax.numpy as jnp
from jax import lax
from jax.experimental import pallas as pl
from jax.experimental.pallas import tpu as pltpu
from jax.experimental.pallas import tpu_sc as plsc

D_MODEL = 64
SCALE = math.exp(64)
LANES = 16

_INFO = plsc.get_sparse_core_info()
NC = _INFO.num_cores
NS = _INFO.num_subcores
NW = NC * NS
SUB = 128
DBLK = D_MODEL // 8


def _make_transpose(v: int):
    vp = (v + SUB - 1) // SUB * SUB
    n_blocks = vp // SUB
    pairs = vp // 2

    mesh = plsc.VectorSubcoreMesh(core_axis_name="c", subcore_axis_name="s")

    @pl.kernel(
        out_type=jax.ShapeDtypeStruct((pairs, 2 * D_MODEL), jnp.float32),
        mesh=mesh,
        scratch_types=[
            pltpu.VMEM((D_MODEL, SUB), jnp.float32),
            pltpu.VMEM((D_MODEL, SUB), jnp.float32),
            pltpu.VMEM((D_MODEL, 2 * D_MODEL + 1), jnp.float32),
            pltpu.VMEM((D_MODEL, 2 * D_MODEL + 1), jnp.float32),
            pltpu.SemaphoreType.DMA,
            pltpu.SemaphoreType.DMA,
            pltpu.SemaphoreType.DMA,
            pltpu.SemaphoreType.DMA,
        ],
        compiler_params=pltpu.CompilerParams(
            use_tc_tiling_on_sc=True,
            needs_layout_passes=False,
            disable_bounds_checks=True,
        ),
    )
    def transpose(tt_hbm, out_hbm, in0, in1, out0, out1, si0, si1, so0, so1):
        wid = lax.axis_index("s") * NC + lax.axis_index("c")
        trips = (n_blocks + NW - 1) // NW
        bufs = ((in0, out0, si0, so0), (in1, out1, si1, so1))
        lane = jax.lax.iota(jnp.int32, LANES)
        rows_k = [lane + LANES * k for k in range(SUB // LANES)]
        prow_k = [(lane + LANES * k) >> 1 for k in range(SUB // LANES)]
        m_even = (lane & 1) == 0
        m_odd = (lane & 1) == 1

        def blk(t):
            return jnp.minimum(wid + NW * t, n_blocks - 1)

        def fire_in(t, in_b, sem):
            c0 = blk(t) * SUB
            pltpu.async_copy(tt_hbm.at[:, pl.ds(c0, SUB)], in_b, sem)

        def wait_in(in_b, sem):
            pltpu.make_async_copy(
                tt_hbm.at[:, pl.ds(0, SUB)], in_b, sem).wait()

        def fire_out(t, out_b, sem):
            p0 = blk(t) * (SUB // 2)
            pltpu.async_copy(
                out_b.at[:, pl.ds(0, 2 * D_MODEL)],
                out_hbm.at[pl.ds(p0, SUB // 2)], sem)

        def wait_out(out_b, sem):
            pltpu.make_async_copy(
                out_b.at[:, pl.ds(0, 2 * D_MODEL)],
                out_hbm.at[pl.ds(0, SUB // 2)], sem).wait()

        def do_transpose(in_b, out_b):
            @plsc.parallel_loop(0, D_MODEL, 1, unroll=2)
            def _(j):
                col = rows_k[0] * 0 + j
                col2 = col + D_MODEL
                for k in range(SUB // LANES):
                    v = in_b[j, pl.ds(LANES * k, LANES)]
                    plsc.store_scatter(
                        out_b, [prow_k[k], col], v, mask=m_even)
                    plsc.store_scatter(
                        out_b, [prow_k[k], col2], v, mask=m_odd)

        for bi in range(2):
            fire_in(bi, bufs[bi][0], bufs[bi][2])
        for bi in range(2):
            in_b, out_b, si, so = bufs[bi]
            wait_in(in_b, si)
            do_transpose(in_b, out_b)
            fire_out(bi, out_b, so)
            fire_in(bi + 2, in_b, si)

        def body(i, _):
            for bi in range(2):
                t = 2 + 2 * i + bi
                in_b, out_b, si, so = bufs[bi]
                wait_in(in_b, si)
                wait_out(out_b, so)
                do_transpose(in_b, out_b)
                fire_out(t, out_b, so)
                fire_in(t + 2, in_b, si)
            return 0

        lax.fori_loop(0, (trips - 4) // 2, body, 0)

        t0 = (trips - 4) // 2 * 2 + 2
        for t in range(t0, trips):
            in_b, out_b, si, so = bufs[t % 2]
            wait_in(in_b, si)
            wait_out(out_b, so)
            do_transpose(in_b, out_b)
            fire_out(t, out_b, so)
            if t + 2 < trips:
                fire_in(t + 2, in_b, si)
        for bi in range(2):
            wait_out(bufs[bi][1], bufs[bi][3])

    return transpose


def _make_lookup(n_items: int, s_total: int, n_bblk: int):
    items_per_w = n_items // NW
    assert items_per_w >= 4 and items_per_w % 2 == 0

    mesh = plsc.VectorSubcoreMesh(core_axis_name="c", subcore_axis_name="s")

    @pl.kernel(
        out_type=jax.ShapeDtypeStruct(
            (s_total, DBLK, n_bblk, 8, SUB), jnp.float32),
        mesh=mesh,
        scratch_types=[
            pltpu.VMEM((items_per_w, SUB), jnp.int32),
            pltpu.VMEM((SUB, D_MODEL), jnp.float32),
            pltpu.VMEM((SUB, D_MODEL), jnp.float32),
            pltpu.VMEM((D_MODEL, SUB + 1), jnp.float32),
            pltpu.VMEM((D_MODEL, SUB + 1), jnp.float32),
            pltpu.SemaphoreType.DMA,
            pltpu.SemaphoreType.DMA,
            pltpu.SemaphoreType.DMA,
            pltpu.SemaphoreType.DMA,
        ],
        compiler_params=pltpu.CompilerParams(
            use_tc_tiling_on_sc=False, needs_layout_passes=False),
    )
    def lookup(idx_hbm, table_hbm, out_hbm, idx_v, in0, in1, out0, out1,
               si0, si1, so0, so1):
        wid = lax.axis_index("s") * NC + lax.axis_index("c")
        u0 = wid * items_per_w
        bufs = ((in0, out0, si0, so0), (in1, out1, si1, so1))
        lane = jax.lax.iota(jnp.int32, LANES)
        rows_k = [lane + LANES * k for k in range(SUB // LANES)]

        def fire_gather(ul, in_b, sem):
            pltpu.async_copy(table_hbm.at[idx_v.at[ul]], in_b, sem)

        def wait_gather(in_b, sem):
            pltpu.make_async_copy(
                table_hbm.at[idx_v.at[0]], in_b, sem).wait()

        def fire_out(u, out_b, sem):
            s = (u // (8 * n_bblk)) * 8 + u % 8
            bblk = (u // 8) % n_bblk
            for j in range(DBLK):
                pltpu.async_copy(
                    out_b.at[pl.ds(8 * j, 8), pl.ds(0, SUB)],
                    out_hbm.at[s, j, bblk], sem)

        def wait_out(out_b, sem):
            for j in range(DBLK):
                pltpu.make_async_copy(
                    out_b.at[pl.ds(8 * j, 8), pl.ds(0, SUB)],
                    out_hbm.at[0, j, 0], sem
                ).wait()

        def transpose_scale(in_b, out_b):
            @plsc.parallel_loop(0, SUB, 1, unroll=2)
            def _(r):
                col = rows_k[0] * 0 + r
                for k in range(D_MODEL // LANES):
                    v = in_b[r, pl.ds(LANES * k, LANES)]
                    plsc.store_scatter(
                        out_b, [rows_k[k], col], v * SCALE)

        pltpu.sync_copy(idx_hbm.at[pl.ds(u0, items_per_w)], idx_v)

        for bi in range(2):
            fire_gather(bi, bufs[bi][0], bufs[bi][2])

        for bi in range(2):
            in_b, out_b, si, so = bufs[bi]
            wait_gather(in_b, si)
            transpose_scale(in_b, out_b)
            fire_out(u0 + bi, out_b, so)
            fire_gather(bi + 2, in_b, si)

        def body(i, _):
            for bi in range(2):
                ul = 2 + 2 * i + bi
                in_b, out_b, si, so = bufs[bi]
                wait_gather(in_b, si)
                wait_out(out_b, so)
                transpose_scale(in_b, out_b)
                fire_out(u0 + ul, out_b, so)
                fire_gather(ul + 2, in_b, si)
            return 0

        lax.fori_loop(0, (items_per_w - 4) // 2, body, 0)

        for bi in range(2):
            ul = items_per_w - 2 + bi
            in_b, out_b, si, so = bufs[bi]
            wait_gather(in_b, si)
            wait_out(out_b, so)
            transpose_scale(in_b, out_b)
            fire_out(u0 + ul, out_b, so)
        for bi in range(2):
            wait_out(bufs[bi][1], bufs[bi][3])

    return lookup


def kernel(x, table):
    b, s = x.shape
    n_bblk = b // SUB
    s_rows = s // 8
    xb = (
        x.T.astype(jnp.int32)
        .reshape(s_rows, 8, n_bblk, SUB)
        .transpose(0, 2, 1, 3)
        .reshape(s_rows * n_bblk * 8, SUB)
    )
    vp = (table.shape[0] + SUB - 1) // SUB * SUB
    pairs = _make_transpose(table.shape[0])(table.T)
    table_rm = pairs.reshape(vp, D_MODEL)
    out5 = _make_lookup(xb.shape[0], s, n_bblk)(xb, table_rm)
    return out5.transpose(2, 4, 0, 1, 3).reshape(b, s, D_MODEL)


# speedup vs baseline: 1.1791x; 1.1791x over previous
import math

import jax
import jax.numpy as jnp
from jax import lax
from jax.experimental import pallas as pl
from jax.experimental.pallas import tpu as pltpu
from jax.experimental.pallas import tpu_sc as plsc

D_MODEL = 64
SCALE = math.exp(64)
LANES = 16

_INFO = plsc.get_sparse_core_info()
NC = _INFO.num_cores
NS = _INFO.num_subcores
NW = NC * NS
SUB = 128
DBLK = D_MODEL // 8


def _make_lookup(n_items: int, s_total: int, n_bblk: int):
    items_per_w = n_items // NW
    assert items_per_w >= 4 and items_per_w % 2 == 0

    mesh = plsc.VectorSubcoreMesh(core_axis_name="c", subcore_axis_name="s")

    @pl.kernel(
        out_type=jax.ShapeDtypeStruct(
            (s_total, DBLK, n_bblk, 8, SUB), jnp.float32),
        mesh=mesh,
        scratch_types=[
            pltpu.VMEM((items_per_w, SUB), jnp.int32),
            pltpu.VMEM((SUB, D_MODEL), jnp.float32),
            pltpu.VMEM((SUB, D_MODEL), jnp.float32),
            pltpu.VMEM((D_MODEL, SUB + 1), jnp.float32),
            pltpu.VMEM((D_MODEL, SUB + 1), jnp.float32),
            pltpu.SemaphoreType.DMA,
            pltpu.SemaphoreType.DMA,
            pltpu.SemaphoreType.DMA,
            pltpu.SemaphoreType.DMA,
        ],
        compiler_params=pltpu.CompilerParams(
            use_tc_tiling_on_sc=False, needs_layout_passes=False),
    )
    def lookup(idx_hbm, table_hbm, out_hbm, idx_v, in0, in1, out0, out1,
               si0, si1, so0, so1):
        wid = lax.axis_index("s") * NC + lax.axis_index("c")
        u0 = wid * items_per_w
        bufs = ((in0, out0, si0, so0), (in1, out1, si1, so1))
        lane = jax.lax.iota(jnp.int32, LANES)
        rows_k = [lane + LANES * k for k in range(SUB // LANES)]

        def fire_gather(ul, in_b, sem):
            pltpu.async_copy(table_hbm.at[idx_v.at[ul]], in_b, sem)

        def wait_gather(in_b, sem):
            pltpu.make_async_copy(
                table_hbm.at[idx_v.at[0]], in_b, sem).wait()

        def fire_out(u, out_b, sem):
            s = (u // (8 * n_bblk)) * 8 + u % 8
            bblk = (u // 8) % n_bblk
            for j in range(DBLK):
                pltpu.async_copy(
                    out_b.at[pl.ds(8 * j, 8), pl.ds(0, SUB)],
                    out_hbm.at[s, j, bblk], sem)

        def wait_out(out_b, sem):
            for j in range(DBLK):
                pltpu.make_async_copy(
                    out_b.at[pl.ds(8 * j, 8), pl.ds(0, SUB)],
                    out_hbm.at[0, j, 0], sem
                ).wait()

        def transpose_scale(in_b, out_b):
            @plsc.parallel_loop(0, SUB, 1, unroll=2)
            def _(r):
                col = rows_k[0] * 0 + r
                for k in range(D_MODEL // LANES):
                    v = in_b[r, pl.ds(LANES * k, LANES)]
                    plsc.store_scatter(
                        out_b, [rows_k[k], col], v * SCALE)

        pltpu.sync_copy(idx_hbm.at[pl.ds(u0, items_per_w)], idx_v)

        for bi in range(2):
            fire_gather(bi, bufs[bi][0], bufs[bi][2])

        for bi in range(2):
            in_b, out_b, si, so = bufs[bi]
            wait_gather(in_b, si)
            transpose_scale(in_b, out_b)
            fire_out(u0 + bi, out_b, so)
            fire_gather(bi + 2, in_b, si)

        def body(i, _):
            for bi in range(2):
                ul = 2 + 2 * i + bi
                in_b, out_b, si, so = bufs[bi]
                wait_gather(in_b, si)
                wait_out(out_b, so)
                transpose_scale(in_b, out_b)
                fire_out(u0 + ul, out_b, so)
                fire_gather(ul + 2, in_b, si)
            return 0

        lax.fori_loop(0, (items_per_w - 4) // 2, body, 0)

        for bi in range(2):
            ul = items_per_w - 2 + bi
            in_b, out_b, si, so = bufs[bi]
            wait_gather(in_b, si)
            wait_out(out_b, so)
            transpose_scale(in_b, out_b)
            fire_out(u0 + ul, out_b, so)
        for bi in range(2):
            wait_out(bufs[bi][1], bufs[bi][3])

    return lookup


def kernel(x, table):
    b, s = x.shape
    n_bblk = b // SUB
    s_rows = s // 8
    xb = (
        x.T.astype(jnp.int32)
        .reshape(s_rows, 8, n_bblk, SUB)
        .transpose(0, 2, 1, 3)
        .reshape(s_rows * n_bblk * 8, SUB)
    )
    out5 = _make_lookup(xb.shape[0], s, n_bblk)(xb, table)
    return out5.transpose(2, 4, 0, 1, 3).reshape(b, s, D_MODEL)


# speedup vs baseline: 1.2393x; 1.0511x over previous
import math

import jax
import jax.numpy as jnp
from jax import lax
from jax.experimental import pallas as pl
from jax.experimental.pallas import tpu as pltpu
from jax.experimental.pallas import tpu_sc as plsc

D_MODEL = 64
SCALE = math.exp(64)
LANES = 16

_INFO = plsc.get_sparse_core_info()
NC = _INFO.num_cores
NS = _INFO.num_subcores
NW = NC * NS
SUB = 128
DBLK = D_MODEL // 8
NBUF = 4


def _make_lookup(n_items: int, s_total: int, n_bblk: int):
    items_per_w = n_items // NW
    assert items_per_w >= 2 * NBUF and items_per_w % NBUF == 0

    mesh = plsc.VectorSubcoreMesh(core_axis_name="c", subcore_axis_name="s")

    @pl.kernel(
        out_type=jax.ShapeDtypeStruct(
            (s_total, DBLK, n_bblk, 8, SUB), jnp.float32),
        mesh=mesh,
        scratch_types=[
            pltpu.VMEM((items_per_w, SUB), jnp.int32),
        ] + [pltpu.VMEM((SUB, D_MODEL), jnp.float32)] * NBUF
          + [pltpu.VMEM((D_MODEL, SUB + 1), jnp.float32)] * NBUF
          + [pltpu.SemaphoreType.DMA] * (2 * NBUF),
        compiler_params=pltpu.CompilerParams(
            use_tc_tiling_on_sc=False, needs_layout_passes=False),
    )
    def lookup(idx_hbm, table_hbm, out_hbm, idx_v, *bref):
        ins, outs = bref[:NBUF], bref[NBUF:2 * NBUF]
        sis = bref[2 * NBUF:3 * NBUF]
        sos = bref[3 * NBUF:4 * NBUF]
        wid = lax.axis_index("s") * NC + lax.axis_index("c")
        u0 = wid * items_per_w
        bufs = tuple(
            (ins[i], outs[i], sis[i], sos[i]) for i in range(NBUF))
        lane = jax.lax.iota(jnp.int32, LANES)
        rows_k = [lane + LANES * k for k in range(SUB // LANES)]

        def fire_gather(ul, in_b, sem):
            pltpu.async_copy(table_hbm.at[idx_v.at[ul]], in_b, sem)

        def wait_gather(in_b, sem):
            pltpu.make_async_copy(
                table_hbm.at[idx_v.at[0]], in_b, sem).wait()

        def fire_out(u, out_b, sem):
            s = (u // (8 * n_bblk)) * 8 + u % 8
            bblk = (u // 8) % n_bblk
            for j in range(DBLK):
                pltpu.async_copy(
                    out_b.at[pl.ds(8 * j, 8), pl.ds(0, SUB)],
                    out_hbm.at[s, j, bblk], sem)

        def wait_out(out_b, sem):
            for j in range(DBLK):
                pltpu.make_async_copy(
                    out_b.at[pl.ds(8 * j, 8), pl.ds(0, SUB)],
                    out_hbm.at[0, j, 0], sem
                ).wait()

        def transpose_scale(in_b, out_b):
            @plsc.parallel_loop(0, SUB, 1, unroll=2)
            def _(r):
                col = rows_k[0] * 0 + r
                for k in range(D_MODEL // LANES):
                    v = in_b[r, pl.ds(LANES * k, LANES)]
                    plsc.store_scatter(
                        out_b, [rows_k[k], col], v * SCALE)

        pltpu.sync_copy(idx_hbm.at[pl.ds(u0, items_per_w)], idx_v)

        for bi in range(NBUF):
            fire_gather(bi, bufs[bi][0], bufs[bi][2])

        for bi in range(NBUF):
            in_b, out_b, si, so = bufs[bi]
            wait_gather(in_b, si)
            transpose_scale(in_b, out_b)
            fire_out(u0 + bi, out_b, so)
            fire_gather(bi + NBUF, in_b, si)

        def body(i, _):
            for bi in range(NBUF):
                ul = NBUF + NBUF * i + bi
                in_b, out_b, si, so = bufs[bi]
                wait_gather(in_b, si)
                wait_out(out_b, so)
                transpose_scale(in_b, out_b)
                fire_out(u0 + ul, out_b, so)
                fire_gather(ul + NBUF, in_b, si)
            return 0

        lax.fori_loop(0, (items_per_w - 2 * NBUF) // NBUF, body, 0)

        for bi in range(NBUF):
            ul = items_per_w - NBUF + bi
            in_b, out_b, si, so = bufs[bi]
            wait_gather(in_b, si)
            wait_out(out_b, so)
            transpose_scale(in_b, out_b)
            fire_out(u0 + ul, out_b, so)
        for bi in range(NBUF):
            wait_out(bufs[bi][1], bufs[bi][3])

    return lookup


def kernel(x, table):
    b, s = x.shape
    n_bblk = b // SUB
    s_rows = s // 8
    xb = (
        x.T.astype(jnp.int32)
        .reshape(s_rows, 8, n_bblk, SUB)
        .transpose(0, 2, 1, 3)
        .reshape(s_rows * n_bblk * 8, SUB)
    )
    out5 = _make_lookup(xb.shape[0], s, n_bblk)(xb, table)
    return out5.transpose(2, 4, 0, 1, 3).reshape(b, s, D_MODEL)
